# baseline (speedup 1.0000x reference)
import functools

import jax
import jax.numpy as jnp
from jax import lax
from jax.experimental import pallas as pl
from jax.experimental.pallas import tpu as pltpu
from jax.experimental.pallas import tpu_sc as plsc

_NT = 16
_NC = 2
_CH = 128


def _iota16():
    return lax.iota(jnp.int32, 16)


def _full16(v):
    return jnp.full((16,), v, dtype=jnp.int32)


def _rsqrt16(x):
    i = plsc.bitcast(x, jnp.int32)
    i = jnp.int32(0x5F3759DF) - (i >> 1)
    y = plsc.bitcast(i, jnp.float32)
    for _ in range(3):
        y = y * (1.5 - 0.5 * x * y * y)
    return y


def _make_prep(n_pad, n_edges):
    rows_per_tile = n_pad // _NT
    eb_deg = n_edges // _NT
    eb_norm = n_edges // (_NT * _NC)
    mesh = plsc.VectorSubcoreMesh(
        core_axis_name="c", subcore_axis_name="s",
        num_cores=_NC, num_subcores=_NT)

    @functools.partial(
        pl.kernel,
        out_type=jax.ShapeDtypeStruct((n_edges,), jnp.float32),
        mesh=mesh,
        compiler_params=pltpu.CompilerParams(
            needs_layout_passes=False, use_tc_tiling_on_sc=False),
        scratch_types=[
            pltpu.VMEM_SHARED((n_pad, 16), jnp.float32),
            pltpu.VMEM_SHARED((n_pad,), jnp.float32),
            pltpu.VMEM((_CH,), jnp.float32),
            pltpu.VMEM((_CH,), jnp.int32),
            pltpu.VMEM((_CH,), jnp.int32),
            pltpu.VMEM((_CH, 16), jnp.float32),
            pltpu.VMEM((rows_per_tile, 16), jnp.float32),
            pltpu.VMEM((rows_per_tile,), jnp.float32),
            pltpu.VMEM((n_pad,), jnp.float32),
            pltpu.VMEM((_CH,), jnp.float32),
        ],
    )
    def prep(src_hbm, dst_hbm, ew_hbm, zero_hbm, norm_hbm,
             acc_sh, dinv_sh, ew_ch, dst_ch, src_ch, row_buf,
             deg_loc, dinv_loc, dinv_full, norm_buf):
        c = lax.axis_index("c")
        s = lax.axis_index("s")
        n0 = s * rows_per_tile

        for r in range(rows_per_tile // _CH):
            pltpu.sync_copy(zero_hbm.at[pl.ds(0, _CH)],
                            acc_sh.at[pl.ds(n0 + r * _CH, _CH)])
        plsc.subcore_barrier()

        base = s * eb_deg

        @pl.loop(0, eb_deg // _CH)
        def _(ch):
            b = base + ch * _CH
            pltpu.sync_copy(ew_hbm.at[pl.ds(b, _CH)], ew_ch)
            pltpu.sync_copy(dst_hbm.at[pl.ds(b, _CH)], dst_ch)

            @pl.loop(0, _CH)
            def _(e):
                row_buf[e, :] = plsc.load_gather(ew_ch, [_full16(e)])

            pltpu.sync_copy(row_buf, acc_sh.at[dst_ch], add=True)

        plsc.subcore_barrier()

        pltpu.sync_copy(acc_sh.at[pl.ds(n0, rows_per_tile)], deg_loc)

        @pl.loop(0, rows_per_tile // 16)
        def _(i):
            idx = i * 16 + _iota16()
            deg16 = plsc.load_gather(deg_loc, [idx, _full16(0)])
            y = _rsqrt16(deg16)
            dinv_loc[pl.ds(i * 16, 16)] = jnp.where(deg16 > 0.0, y, 0.0)

        pltpu.sync_copy(dinv_loc, dinv_sh.at[pl.ds(n0, rows_per_tile)])
        plsc.subcore_barrier()
        pltpu.sync_copy(dinv_sh, dinv_full)

        wid = s * _NC + c
        base2 = wid * eb_norm

        def norm_chunk(b, length):
            pltpu.sync_copy(src_hbm.at[pl.ds(b, length)],
                            src_ch.at[pl.ds(0, length)])
            pltpu.sync_copy(dst_hbm.at[pl.ds(b, length)],
                            dst_ch.at[pl.ds(0, length)])
            pltpu.sync_copy(ew_hbm.at[pl.ds(b, length)],
                            ew_ch.at[pl.ds(0, length)])
            for g in range(length // 16):
                s16 = src_ch[pl.ds(g * 16, 16)]
                d16 = dst_ch[pl.ds(g * 16, 16)]
                w16 = ew_ch[pl.ds(g * 16, 16)]
                ds_ = plsc.load_gather(dinv_full, [s16])
                dd_ = plsc.load_gather(dinv_full, [d16])
                norm_buf[pl.ds(g * 16, 16)] = ds_ * w16 * dd_
            pltpu.sync_copy(norm_buf.at[pl.ds(0, length)],
                            norm_hbm.at[pl.ds(b, length)])

        nfull = eb_norm // _CH
        tail = eb_norm % _CH

        @pl.loop(0, nfull)
        def _(ch):
            norm_chunk(base2 + ch * _CH, _CH)

        if tail:
            norm_chunk(base2 + nfull * _CH, tail)

    return prep


def _make_layer_props(n_pad, n_chunks, d):
    dh = 64
    npass = d // (2 * dh)
    a = d // dh
    shift = a.bit_length() - 1
    rows_per_tile = n_pad // _NT
    nch = n_chunks // _NT
    assert nch >= 4 and nch % 2 == 1
    out_sds = jax.ShapeDtypeStruct((a * n_pad, dh), jnp.float32)
    mesh = plsc.VectorSubcoreMesh(
        core_axis_name="c", subcore_axis_name="s",
        num_cores=_NC, num_subcores=_NT)

    @functools.partial(
        pl.kernel,
        out_type=(out_sds, out_sds, out_sds),
        mesh=mesh,
        compiler_params=pltpu.CompilerParams(
            needs_layout_passes=False, use_tc_tiling_on_sc=False),
        scratch_types=[
            pltpu.VMEM_SHARED((n_pad, dh), jnp.float32),
            pltpu.VMEM((_CH, dh), jnp.float32),
            pltpu.VMEM((_CH, dh), jnp.float32),
            pltpu.VMEM((nch * _CH,), jnp.int32),
            pltpu.VMEM((nch, _CH), jnp.int32),
            pltpu.VMEM((nch * _CH,), jnp.int32),
            pltpu.SemaphoreType.DMA,
            pltpu.SemaphoreType.DMA,
            pltpu.SemaphoreType.DMA,
            pltpu.SemaphoreType.DMA,
        ],
    )
    def props(table0, edges_hbm, zero_hbm,
              o1, o2, o3,
              acc, rows0, rows1, gidx, dstb, normb, sg0, sg1, ss0, ss1):
        c = lax.axis_index("c")
        s = lax.axis_index("s")
        n0 = s * rows_per_tile
        eb = s * (nch * _CH)
        bufs = (rows0, rows1)
        gsems = (sg0, sg1)
        ssems = (ss0, ss1)

        def zero_acc():
            for r in range(rows_per_tile // _CH):
                pltpu.sync_copy(zero_hbm.at[pl.ds(0, _CH)],
                                acc.at[pl.ds(n0 + r * _CH, _CH)])

        pltpu.sync_copy(edges_hbm.at[1, pl.ds(eb, nch * _CH)], normb)

        @pl.loop(0, nch)
        def _(ch):
            for g in range(_CH // 16):
                dstb[ch, pl.ds(g * 16, 16)] = normb[pl.ds(ch * _CH + g * 16, 16)]

        pltpu.sync_copy(edges_hbm.at[0, pl.ds(eb, nch * _CH)], gidx)
        pltpu.sync_copy(edges_hbm.at[2, pl.ds(eb, nch * _CH)], normb)

        @pl.loop(0, nch * (_CH // 16))
        def _(i):
            v = gidx[pl.ds(i * 16, 16)]
            gidx[pl.ds(i * 16, 16)] = v * a + c

        def transform(fn):
            @pl.loop(0, nch * (_CH // 16))
            def _(i):
                v = gidx[pl.ds(i * 16, 16)]
                gidx[pl.ds(i * 16, 16)] = fn(v)

        zero_acc()
        plsc.subcore_barrier()

        def issue_gather(table_ref, ch, b):
            pltpu.async_copy(table_ref.at[gidx.at[pl.ds(ch * _CH, _CH)]],
                             bufs[b], gsems[b])

        def wait_gather(table_ref, ch, b):
            pltpu.make_async_copy(
                table_ref.at[gidx.at[pl.ds(ch * _CH, _CH)]],
                bufs[b], gsems[b]).wait()

        def issue_scatter(ch, b):
            pltpu.async_copy(bufs[b], acc.at[dstb.at[ch]], ssems[b], add=True)

        def wait_scatter(ch, b):
            pltpu.make_async_copy(
                bufs[b], acc.at[dstb.at[ch]], ssems[b]).wait()

        def scale(ch, b):
            rows = bufs[b]

            @pl.loop(0, _CH // 16)
            def _(g):
                nf16 = plsc.bitcast(
                    normb[pl.ds(ch * _CH + g * 16, 16)], jnp.float32)
                for j in range(16):
                    e = g * 16 + j
                    nb = jnp.full((16,), nf16[j], dtype=jnp.float32)
                    for q in range(dh // 16):
                        rows[e, pl.ds(q * 16, 16)] = (
                            rows[e, pl.ds(q * 16, 16)] * nb)

        def run_pass(table_ref, out_ref, q_off):
            issue_gather(table_ref, 0, 0)

            @pl.loop(0, (nch + 1) // 2)
            def _(p):
                ch0 = 2 * p
                wait_gather(table_ref, ch0, 0)
                scale(ch0, 0)

                @pl.when(ch0 >= 1)
                def _():
                    wait_scatter(ch0 - 1, 1)

                @pl.when(ch0 + 1 < nch)
                def _():
                    issue_gather(table_ref, ch0 + 1, 1)

                issue_scatter(ch0, 0)
                ch1 = ch0 + 1

                @pl.when(ch1 < nch)
                def _():
                    wait_gather(table_ref, ch1, 1)
                    scale(ch1, 1)
                    wait_scatter(ch1 - 1, 0)

                    @pl.when(ch1 + 1 < nch)
                    def _():
                        issue_gather(table_ref, ch1 + 1, 0)

                    issue_scatter(ch1, 1)

            wait_scatter(nch - 1, 0)
            plsc.subcore_barrier()
            pltpu.sync_copy(acc.at[pl.ds(n0, rows_per_tile)],
                            out_ref.at[pl.ds(q_off + n0, rows_per_tile)])
            zero_acc()
            plsc.subcore_barrier()

        run_pass(table0, o1, c * n_pad)
        if npass == 2:
            transform(lambda v: v + 2)
            run_pass(table0, o1, (c + 2) * n_pad)

        sub = 2 * (npass - 1)
        transform(lambda v: ((v - c - sub) >> shift) + c * n_pad)
        for tbl, out in ((o1, o2), (o2, o3)):
            run_pass(tbl, out, c * n_pad)
            if npass == 2:
                transform(lambda v: v + 2 * n_pad)
                run_pass(tbl, out, (c + 2) * n_pad)
                if out is not o3:
                    transform(lambda v: v - 2 * n_pad)

    return props


def _mm_layer(h0, p1, p2, p3, w, b, relu):
    n_pad, d_in = h0.shape
    d_out = w.shape[2]
    a = p1.shape[0]
    bm = 1280
    grid = (n_pad // bm,)

    def body(h0_ref, p1_ref, p2_ref, p3_ref, w_ref, b_ref, o_ref):
        dot = functools.partial(
            jnp.dot, preferred_element_type=jnp.float32,
            precision=jax.lax.Precision.HIGHEST)
        acc = dot(h0_ref[...], w_ref[0])
        for k, pr in ((1, p1_ref), (2, p2_ref), (3, p3_ref)):
            for q in range(a):
                acc += dot(pr[q], w_ref[k, q * 64:(q + 1) * 64, :])
        acc = acc + b_ref[...]
        if relu:
            acc = jnp.maximum(acc, 0.0)
        o_ref[...] = acc

    return pl.pallas_call(
        body,
        grid=grid,
        in_specs=[
            pl.BlockSpec((bm, d_in), lambda i: (i, 0)),
            pl.BlockSpec((a, bm, 64), lambda i: (0, i, 0)),
            pl.BlockSpec((a, bm, 64), lambda i: (0, i, 0)),
            pl.BlockSpec((a, bm, 64), lambda i: (0, i, 0)),
            pl.BlockSpec((4, d_in, d_out), lambda i: (0, 0, 0)),
            pl.BlockSpec((1, d_out), lambda i: (0, 0)),
        ],
        out_specs=pl.BlockSpec((bm, d_out), lambda i: (i, 0)),
        out_shape=jax.ShapeDtypeStruct((n_pad, d_out), jnp.float32),
    )(h0, p1, p2, p3, w, b.reshape(1, d_out))


def kernel(x, edge_index, edge_attr, W1, b1, W2, b2, W3, b3):
    n, d_in = x.shape
    e = edge_index.shape[1]
    n_pad = -(-n // (_NT * _CH)) * (_NT * _CH)

    src = edge_index[0]
    dst = edge_index[1]
    ew = edge_attr.reshape(-1)
    zero_rows = jnp.zeros((_CH, 16), jnp.float32)

    norm = _make_prep(n_pad, e)(src, dst, ew, zero_rows)

    nch = -(-e // (_NT * _CH))
    if nch % 2 == 0:
        nch += 1
    e_pad = _NT * _CH * nch
    n_chunks = _NT * nch
    edata = jnp.stack([
        jnp.pad(src, (0, e_pad - e)),
        jnp.pad(dst, (0, e_pad - e)),
        lax.bitcast_convert_type(jnp.pad(norm, (0, e_pad - e)), jnp.int32),
    ])

    def layer(h, w, b, relu):
        d = h.shape[1]
        a = d // 64
        zrows = jnp.zeros((_CH, 64), jnp.float32)
        table0 = h.reshape(a * n_pad, 64)
        p1, p2, p3 = _make_layer_props(n_pad, n_chunks, d)(
            table0, edata, zrows)
        return _mm_layer(
            h,
            p1.reshape(a, n_pad, 64),
            p2.reshape(a, n_pad, 64),
            p3.reshape(a, n_pad, 64),
            w, b, relu)

    h = jnp.pad(x, ((0, n_pad - n), (0, 0)))
    h = layer(h, W1, b1, relu=True)
    h = layer(h, W2, b2, relu=True)
    h = layer(h, W3, b3, relu=False)
    return h[:n]


# speedup vs baseline: 1.3910x; 1.3910x over previous
import functools

import jax
import jax.numpy as jnp
from jax import lax
from jax.experimental import pallas as pl
from jax.experimental.pallas import tpu as pltpu
from jax.experimental.pallas import tpu_sc as plsc

_NT = 16
_NC = 2
_CH = 128


def _iota16():
    return lax.iota(jnp.int32, 16)


def _full16(v):
    return jnp.full((16,), v, dtype=jnp.int32)


def _rsqrt16(x):
    i = plsc.bitcast(x, jnp.int32)
    i = jnp.int32(0x5F3759DF) - (i >> 1)
    y = plsc.bitcast(i, jnp.float32)
    for _ in range(3):
        y = y * (1.5 - 0.5 * x * y * y)
    return y


def _make_prep(n_pad, n_edges):
    rows_per_tile = n_pad // _NT
    eb_deg = n_edges // _NT
    eb_norm = n_edges // (_NT * _NC)
    mesh = plsc.VectorSubcoreMesh(
        core_axis_name="c", subcore_axis_name="s",
        num_cores=_NC, num_subcores=_NT)

    @functools.partial(
        pl.kernel,
        out_type=jax.ShapeDtypeStruct((n_edges,), jnp.float32),
        mesh=mesh,
        compiler_params=pltpu.CompilerParams(
            needs_layout_passes=False, use_tc_tiling_on_sc=False),
        scratch_types=[
            pltpu.VMEM_SHARED((n_pad, 16), jnp.float32),
            pltpu.VMEM_SHARED((n_pad,), jnp.float32),
            pltpu.VMEM((_CH,), jnp.float32),
            pltpu.VMEM((_CH,), jnp.int32),
            pltpu.VMEM((_CH,), jnp.int32),
            pltpu.VMEM((_CH, 16), jnp.float32),
            pltpu.VMEM((rows_per_tile, 16), jnp.float32),
            pltpu.VMEM((rows_per_tile,), jnp.float32),
            pltpu.VMEM((n_pad,), jnp.float32),
            pltpu.VMEM((_CH,), jnp.float32),
        ],
    )
    def prep(src_hbm, dst_hbm, ew_hbm, zero_hbm, norm_hbm,
             acc_sh, dinv_sh, ew_ch, dst_ch, src_ch, row_buf,
             deg_loc, dinv_loc, dinv_full, norm_buf):
        c = lax.axis_index("c")
        s = lax.axis_index("s")
        n0 = s * rows_per_tile

        for r in range(rows_per_tile // _CH):
            pltpu.sync_copy(zero_hbm.at[pl.ds(0, _CH)],
                            acc_sh.at[pl.ds(n0 + r * _CH, _CH)])
        plsc.subcore_barrier()

        base = s * eb_deg

        @pl.loop(0, eb_deg // _CH)
        def _(ch):
            b = base + ch * _CH
            pltpu.sync_copy(ew_hbm.at[pl.ds(b, _CH)], ew_ch)
            pltpu.sync_copy(dst_hbm.at[pl.ds(b, _CH)], dst_ch)

            @pl.loop(0, _CH)
            def _(e):
                row_buf[e, :] = plsc.load_gather(ew_ch, [_full16(e)])

            pltpu.sync_copy(row_buf, acc_sh.at[dst_ch], add=True)

        plsc.subcore_barrier()

        pltpu.sync_copy(acc_sh.at[pl.ds(n0, rows_per_tile)], deg_loc)

        @pl.loop(0, rows_per_tile // 16)
        def _(i):
            idx = i * 16 + _iota16()
            deg16 = plsc.load_gather(deg_loc, [idx, _full16(0)])
            y = _rsqrt16(deg16)
            dinv_loc[pl.ds(i * 16, 16)] = jnp.where(deg16 > 0.0, y, 0.0)

        pltpu.sync_copy(dinv_loc, dinv_sh.at[pl.ds(n0, rows_per_tile)])
        plsc.subcore_barrier()
        pltpu.sync_copy(dinv_sh, dinv_full)

        wid = s * _NC + c
        base2 = wid * eb_norm

        def norm_chunk(b, length):
            pltpu.sync_copy(src_hbm.at[pl.ds(b, length)],
                            src_ch.at[pl.ds(0, length)])
            pltpu.sync_copy(dst_hbm.at[pl.ds(b, length)],
                            dst_ch.at[pl.ds(0, length)])
            pltpu.sync_copy(ew_hbm.at[pl.ds(b, length)],
                            ew_ch.at[pl.ds(0, length)])
            for g in range(length // 16):
                s16 = src_ch[pl.ds(g * 16, 16)]
                d16 = dst_ch[pl.ds(g * 16, 16)]
                w16 = ew_ch[pl.ds(g * 16, 16)]
                ds_ = plsc.load_gather(dinv_full, [s16])
                dd_ = plsc.load_gather(dinv_full, [d16])
                norm_buf[pl.ds(g * 16, 16)] = ds_ * w16 * dd_
            pltpu.sync_copy(norm_buf.at[pl.ds(0, length)],
                            norm_hbm.at[pl.ds(b, length)])

        nfull = eb_norm // _CH
        tail = eb_norm % _CH

        @pl.loop(0, nfull)
        def _(ch):
            norm_chunk(base2 + ch * _CH, _CH)

        if tail:
            norm_chunk(base2 + nfull * _CH, tail)

    return prep


def _make_layer_props(n_pad, n_chunks, d):
    dh = 64
    npass = d // (2 * dh)
    a = d // dh
    shift = a.bit_length() - 1
    rows_per_tile = n_pad // _NT
    nch = n_chunks // _NT
    assert nch >= 4
    out_sds = jax.ShapeDtypeStruct((a * n_pad, dh), jnp.float32)
    mesh = plsc.VectorSubcoreMesh(
        core_axis_name="c", subcore_axis_name="s",
        num_cores=_NC, num_subcores=_NT)

    @functools.partial(
        pl.kernel,
        out_type=(out_sds, out_sds, out_sds),
        mesh=mesh,
        compiler_params=pltpu.CompilerParams(
            needs_layout_passes=False, use_tc_tiling_on_sc=False),
        scratch_types=[
            pltpu.VMEM_SHARED((n_pad, dh), jnp.float32),
            pltpu.VMEM((_CH, dh), jnp.float32),
            pltpu.VMEM((_CH, dh), jnp.float32),
            pltpu.VMEM((_CH, dh), jnp.float32),
            pltpu.VMEM((nch * _CH,), jnp.int32),
            pltpu.VMEM((nch, _CH), jnp.int32),
            pltpu.VMEM((nch * _CH,), jnp.int32),
            pltpu.SemaphoreType.DMA,
            pltpu.SemaphoreType.DMA,
            pltpu.SemaphoreType.DMA,
            pltpu.SemaphoreType.DMA,
            pltpu.SemaphoreType.DMA,
            pltpu.SemaphoreType.DMA,
        ],
    )
    def props(table0, edges_hbm, zero_hbm,
              o1, o2, o3,
              acc, rows0, rows1, rows2, gidx, dstb, normb,
              sg0, sg1, sg2, ss0, ss1, ss2):
        c = lax.axis_index("c")
        s = lax.axis_index("s")
        n0 = s * rows_per_tile
        eb = s * (nch * _CH)
        bufs = (rows0, rows1, rows2)
        gsems = (sg0, sg1, sg2)
        ssems = (ss0, ss1, ss2)

        def zero_acc():
            for r in range(rows_per_tile // _CH):
                pltpu.sync_copy(zero_hbm.at[pl.ds(0, _CH)],
                                acc.at[pl.ds(n0 + r * _CH, _CH)])

        pltpu.sync_copy(edges_hbm.at[1, pl.ds(eb, nch * _CH)], normb)

        @pl.loop(0, nch)
        def _(ch):
            for g in range(_CH // 16):
                dstb[ch, pl.ds(g * 16, 16)] = normb[pl.ds(ch * _CH + g * 16, 16)]

        pltpu.sync_copy(edges_hbm.at[0, pl.ds(eb, nch * _CH)], gidx)
        pltpu.sync_copy(edges_hbm.at[2, pl.ds(eb, nch * _CH)], normb)

        @pl.loop(0, nch * (_CH // 16))
        def _(i):
            v = gidx[pl.ds(i * 16, 16)]
            gidx[pl.ds(i * 16, 16)] = v * a + c

        def transform(fn):
            @pl.loop(0, nch * (_CH // 16))
            def _(i):
                v = gidx[pl.ds(i * 16, 16)]
                gidx[pl.ds(i * 16, 16)] = fn(v)

        zero_acc()
        plsc.subcore_barrier()

        def issue_gather(table_ref, ch, b):
            pltpu.async_copy(table_ref.at[gidx.at[pl.ds(ch * _CH, _CH)]],
                             bufs[b], gsems[b])

        def wait_gather(table_ref, ch, b):
            pltpu.make_async_copy(
                table_ref.at[gidx.at[pl.ds(ch * _CH, _CH)]],
                bufs[b], gsems[b]).wait()

        def issue_scatter(ch, b):
            pltpu.async_copy(bufs[b], acc.at[dstb.at[ch]], ssems[b], add=True)

        def wait_scatter(ch, b):
            pltpu.make_async_copy(
                bufs[b], acc.at[dstb.at[ch]], ssems[b]).wait()

        def scale(ch, b):
            rows = bufs[b]

            @pl.loop(0, _CH // 16)
            def _(g):
                nf16 = plsc.bitcast(
                    normb[pl.ds(ch * _CH + g * 16, 16)], jnp.float32)
                for j in range(16):
                    e = g * 16 + j
                    nb = jnp.full((16,), nf16[j], dtype=jnp.float32)
                    for q in range(dh // 16):
                        rows[e, pl.ds(q * 16, 16)] = (
                            rows[e, pl.ds(q * 16, 16)] * nb)

        def run_pass(table_ref, out_ref, q_off):
            issue_gather(table_ref, 0, 0)
            issue_gather(table_ref, 1, 1)

            @pl.loop(0, (nch + 2) // 3)
            def _(p):
                for lane in range(3):
                    ch = 3 * p + lane

                    @pl.when(ch < nch)
                    def _():
                        wait_gather(table_ref, ch, lane)
                        scale(ch, lane)

                        @pl.when(ch >= 1)
                        def _():
                            wait_scatter(ch - 1, (lane + 2) % 3)

                        @pl.when(ch + 2 < nch)
                        def _():
                            issue_gather(table_ref, ch + 2, (lane + 2) % 3)

                        issue_scatter(ch, lane)

            wait_scatter(nch - 1, (nch - 1) % 3)
            plsc.subcore_barrier()
            pltpu.sync_copy(acc.at[pl.ds(n0, rows_per_tile)],
                            out_ref.at[pl.ds(q_off + n0, rows_per_tile)])
            zero_acc()
            plsc.subcore_barrier()

        run_pass(table0, o1, c * n_pad)
        if npass == 2:
            transform(lambda v: v + 2)
            run_pass(table0, o1, (c + 2) * n_pad)

        sub = 2 * (npass - 1)
        transform(lambda v: ((v - c - sub) >> shift) + c * n_pad)
        for tbl, out in ((o1, o2), (o2, o3)):
            run_pass(tbl, out, c * n_pad)
            if npass == 2:
                transform(lambda v: v + 2 * n_pad)
                run_pass(tbl, out, (c + 2) * n_pad)
                if out is not o3:
                    transform(lambda v: v - 2 * n_pad)

    return props


def _mm_layer(h0, p1, p2, p3, w, b, relu):
    n_pad, d_in = h0.shape
    d_out = w.shape[2]
    a = p1.shape[0]
    bm = 1280
    grid = (n_pad // bm,)

    def body(h0_ref, p1_ref, p2_ref, p3_ref, w_ref, b_ref, o_ref):
        dot = functools.partial(
            jnp.dot, preferred_element_type=jnp.float32,
            precision=jax.lax.Precision.HIGHEST)
        acc = dot(h0_ref[...], w_ref[0])
        for k, pr in ((1, p1_ref), (2, p2_ref), (3, p3_ref)):
            for q in range(a):
                acc += dot(pr[q], w_ref[k, q * 64:(q + 1) * 64, :])
        acc = acc + b_ref[...]
        if relu:
            acc = jnp.maximum(acc, 0.0)
        o_ref[...] = acc

    return pl.pallas_call(
        body,
        grid=grid,
        in_specs=[
            pl.BlockSpec((bm, d_in), lambda i: (i, 0)),
            pl.BlockSpec((a, bm, 64), lambda i: (0, i, 0)),
            pl.BlockSpec((a, bm, 64), lambda i: (0, i, 0)),
            pl.BlockSpec((a, bm, 64), lambda i: (0, i, 0)),
            pl.BlockSpec((4, d_in, d_out), lambda i: (0, 0, 0)),
            pl.BlockSpec((1, d_out), lambda i: (0, 0)),
        ],
        out_specs=pl.BlockSpec((bm, d_out), lambda i: (i, 0)),
        out_shape=jax.ShapeDtypeStruct((n_pad, d_out), jnp.float32),
    )(h0, p1, p2, p3, w, b.reshape(1, d_out))


def kernel(x, edge_index, edge_attr, W1, b1, W2, b2, W3, b3):
    n, d_in = x.shape
    e = edge_index.shape[1]
    n_pad = -(-n // (_NT * _CH)) * (_NT * _CH)

    src = edge_index[0]
    dst = edge_index[1]
    ew = edge_attr.reshape(-1)
    zero_rows = jnp.zeros((_CH, 16), jnp.float32)

    norm = _make_prep(n_pad, e)(src, dst, ew, zero_rows)

    nch = -(-e // (_NT * _CH))
    if nch % 2 == 0:
        nch += 1
    e_pad = _NT * _CH * nch
    n_chunks = _NT * nch
    edata = jnp.stack([
        jnp.pad(src, (0, e_pad - e)),
        jnp.pad(dst, (0, e_pad - e)),
        lax.bitcast_convert_type(jnp.pad(norm, (0, e_pad - e)), jnp.int32),
    ])

    def layer(h, w, b, relu):
        d = h.shape[1]
        a = d // 64
        zrows = jnp.zeros((_CH, 64), jnp.float32)
        table0 = h.reshape(a * n_pad, 64)
        p1, p2, p3 = _make_layer_props(n_pad, n_chunks, d)(
            table0, edata, zrows)
        return _mm_layer(
            h,
            p1.reshape(a, n_pad, 64),
            p2.reshape(a, n_pad, 64),
            p3.reshape(a, n_pad, 64),
            w, b, relu)

    h = jnp.pad(x, ((0, n_pad - n), (0, 0)))
    h = layer(h, W1, b1, relu=True)
    h = layer(h, W2, b2, relu=True)
    h = layer(h, W3, b3, relu=False)
    return h[:n]


# speedup vs baseline: 1.5158x; 1.0897x over previous
import functools

import jax
import jax.numpy as jnp
from jax import lax
from jax.experimental import pallas as pl
from jax.experimental.pallas import tpu as pltpu
from jax.experimental.pallas import tpu_sc as plsc

_NT = 16
_NC = 2
_CH = 128


def _iota16():
    return lax.iota(jnp.int32, 16)


def _full16(v):
    return jnp.full((16,), v, dtype=jnp.int32)


def _rsqrt16(x):
    i = plsc.bitcast(x, jnp.int32)
    i = jnp.int32(0x5F3759DF) - (i >> 1)
    y = plsc.bitcast(i, jnp.float32)
    for _ in range(3):
        y = y * (1.5 - 0.5 * x * y * y)
    return y


def _make_prep(n_pad, e_pad):
    rows_per_tile = n_pad // _NT
    nchp = e_pad // (_NT * _CH)
    mesh = plsc.VectorSubcoreMesh(
        core_axis_name="c", subcore_axis_name="s",
        num_cores=_NC, num_subcores=_NT)

    @functools.partial(
        pl.kernel,
        out_type=jax.ShapeDtypeStruct((e_pad,), jnp.int32),
        mesh=mesh,
        compiler_params=pltpu.CompilerParams(
            needs_layout_passes=False, use_tc_tiling_on_sc=False),
        scratch_types=[
            pltpu.VMEM_SHARED((n_pad, 16), jnp.float32),
            pltpu.VMEM_SHARED((n_pad,), jnp.float32),
            pltpu.VMEM((nchp * _CH,), jnp.int32),
            pltpu.VMEM((nchp * _CH,), jnp.int32),
            pltpu.VMEM((nchp, _CH), jnp.int32),
            pltpu.VMEM((_CH, 16), jnp.float32),
            pltpu.VMEM((_CH, 16), jnp.float32),
            pltpu.VMEM((_CH, 16), jnp.float32),
            pltpu.VMEM((rows_per_tile, 16), jnp.float32),
            pltpu.VMEM((rows_per_tile,), jnp.float32),
            pltpu.VMEM((n_pad,), jnp.float32),
            pltpu.SemaphoreType.DMA,
            pltpu.SemaphoreType.DMA,
            pltpu.SemaphoreType.DMA,
        ],
    )
    def prep(pdata_hbm, zero_hbm, norm_hbm,
             acc_sh, dinv_sh, ewb, srcb, dstb, rb0, rb1, rb2,
             deg_loc, dinv_loc, dinv_full, ss0, ss1, ss2):
        c = lax.axis_index("c")
        s = lax.axis_index("s")
        n0 = s * rows_per_tile
        eb = s * (nchp * _CH)
        rbufs = (rb0, rb1, rb2)
        ssems = (ss0, ss1, ss2)

        for r in range(rows_per_tile // _CH):
            pltpu.sync_copy(zero_hbm.at[pl.ds(0, _CH)],
                            acc_sh.at[pl.ds(n0 + r * _CH, _CH)])
        pltpu.sync_copy(pdata_hbm.at[1, pl.ds(eb, nchp * _CH)], ewb)

        @pl.loop(0, nchp)
        def _(ch):
            for g in range(_CH // 16):
                dstb[ch, pl.ds(g * 16, 16)] = ewb[pl.ds(ch * _CH + g * 16, 16)]

        pltpu.sync_copy(pdata_hbm.at[0, pl.ds(eb, nchp * _CH)], srcb)
        pltpu.sync_copy(pdata_hbm.at[2, pl.ds(eb, nchp * _CH)], ewb)
        plsc.subcore_barrier()

        def build(ch, b):
            rows = rbufs[b]

            @pl.loop(0, _CH // 16)
            def _(g):
                ew16 = plsc.bitcast(
                    ewb[pl.ds(ch * _CH + g * 16, 16)], jnp.float32)
                for j in range(16):
                    rows[g * 16 + j, :] = jnp.full(
                        (16,), ew16[j], dtype=jnp.float32)

        def issue_s(ch, b):
            pltpu.async_copy(rbufs[b], acc_sh.at[dstb.at[ch]],
                             ssems[b], add=True)

        def wait_s(ch, b):
            pltpu.make_async_copy(
                rbufs[b], acc_sh.at[dstb.at[ch]], ssems[b]).wait()

        @pl.loop(0, (nchp + 2) // 3)
        def _(p):
            for lane in range(3):
                ch = 3 * p + lane

                @pl.when(ch < nchp)
                def _():
                    @pl.when(ch >= 3)
                    def _():
                        wait_s(ch - 3, lane)

                    build(ch, lane)
                    issue_s(ch, lane)

        for k in (3, 2, 1):
            wait_s(nchp - k, (nchp - k) % 3)
        plsc.subcore_barrier()

        pltpu.sync_copy(acc_sh.at[pl.ds(n0, rows_per_tile)], deg_loc)

        @pl.loop(0, rows_per_tile // 16)
        def _(i):
            idx = i * 16 + _iota16()
            deg16 = plsc.load_gather(deg_loc, [idx, _full16(0)])
            y = _rsqrt16(deg16)
            dinv_loc[pl.ds(i * 16, 16)] = jnp.where(deg16 > 0.0, y, 0.0)

        pltpu.sync_copy(dinv_loc, dinv_sh.at[pl.ds(n0, rows_per_tile)])
        plsc.subcore_barrier()
        pltpu.sync_copy(dinv_sh, dinv_full)

        @pl.loop(0, nchp)
        def _(ch):
            for g in range(_CH // 16):
                fl = pl.ds(ch * _CH + g * 16, 16)
                s16 = srcb[fl]
                d16 = dstb[ch, pl.ds(g * 16, 16)]
                w16 = plsc.bitcast(ewb[fl], jnp.float32)
                ds_ = plsc.load_gather(dinv_full, [s16])
                dd_ = plsc.load_gather(dinv_full, [d16])
                ewb[fl] = plsc.bitcast(ds_ * w16 * dd_, jnp.int32)

        @pl.when(c == 0)
        def _():
            piece = (nchp * _CH) // 4
            for i in range(4):
                pltpu.sync_copy(ewb.at[pl.ds(i * piece, piece)],
                                norm_hbm.at[pl.ds(eb + i * piece, piece)])

    return prep


def _make_layer_props(n_pad, n_chunks, d):
    dh = 64
    npass = d // (2 * dh)
    a = d // dh
    shift = a.bit_length() - 1
    rows_per_tile = n_pad // _NT
    nch = n_chunks // _NT
    assert nch >= 4
    out_sds = jax.ShapeDtypeStruct((a * n_pad, dh), jnp.float32)
    mesh = plsc.VectorSubcoreMesh(
        core_axis_name="c", subcore_axis_name="s",
        num_cores=_NC, num_subcores=_NT)

    @functools.partial(
        pl.kernel,
        out_type=(out_sds, out_sds, out_sds),
        mesh=mesh,
        compiler_params=pltpu.CompilerParams(
            needs_layout_passes=False, use_tc_tiling_on_sc=False),
        scratch_types=[
            pltpu.VMEM_SHARED((n_pad, dh), jnp.float32),
            pltpu.VMEM((_CH, dh), jnp.float32),
            pltpu.VMEM((_CH, dh), jnp.float32),
            pltpu.VMEM((_CH, dh), jnp.float32),
            pltpu.VMEM((nch * _CH,), jnp.int32),
            pltpu.VMEM((nch, _CH), jnp.int32),
            pltpu.VMEM((nch * _CH,), jnp.int32),
            pltpu.SemaphoreType.DMA,
            pltpu.SemaphoreType.DMA,
            pltpu.SemaphoreType.DMA,
            pltpu.SemaphoreType.DMA,
            pltpu.SemaphoreType.DMA,
            pltpu.SemaphoreType.DMA,
        ],
    )
    def props(table0, edges_hbm, zero_hbm,
              o1, o2, o3,
              acc, rows0, rows1, rows2, gidx, dstb, normb,
              sg0, sg1, sg2, ss0, ss1, ss2):
        c = lax.axis_index("c")
        s = lax.axis_index("s")
        n0 = s * rows_per_tile
        eb = s * (nch * _CH)
        bufs = (rows0, rows1, rows2)
        gsems = (sg0, sg1, sg2)
        ssems = (ss0, ss1, ss2)

        def zero_acc():
            for r in range(rows_per_tile // _CH):
                pltpu.sync_copy(zero_hbm.at[pl.ds(0, _CH)],
                                acc.at[pl.ds(n0 + r * _CH, _CH)])

        pltpu.sync_copy(edges_hbm.at[1, pl.ds(eb, nch * _CH)], normb)

        @pl.loop(0, nch)
        def _(ch):
            for g in range(_CH // 16):
                dstb[ch, pl.ds(g * 16, 16)] = normb[pl.ds(ch * _CH + g * 16, 16)]

        pltpu.sync_copy(edges_hbm.at[0, pl.ds(eb, nch * _CH)], gidx)
        pltpu.sync_copy(edges_hbm.at[2, pl.ds(eb, nch * _CH)], normb)

        @pl.loop(0, nch * (_CH // 16))
        def _(i):
            v = gidx[pl.ds(i * 16, 16)]
            gidx[pl.ds(i * 16, 16)] = v * a + c

        def transform(fn):
            @pl.loop(0, nch * (_CH // 16))
            def _(i):
                v = gidx[pl.ds(i * 16, 16)]
                gidx[pl.ds(i * 16, 16)] = fn(v)

        zero_acc()
        plsc.subcore_barrier()

        def issue_gather(table_ref, ch, b):
            pltpu.async_copy(table_ref.at[gidx.at[pl.ds(ch * _CH, _CH)]],
                             bufs[b], gsems[b])

        def wait_gather(table_ref, ch, b):
            pltpu.make_async_copy(
                table_ref.at[gidx.at[pl.ds(ch * _CH, _CH)]],
                bufs[b], gsems[b]).wait()

        def issue_scatter(ch, b):
            pltpu.async_copy(bufs[b], acc.at[dstb.at[ch]], ssems[b], add=True)

        def wait_scatter(ch, b):
            pltpu.make_async_copy(
                bufs[b], acc.at[dstb.at[ch]], ssems[b]).wait()

        def scale(ch, b):
            rows = bufs[b]

            @pl.loop(0, _CH // 16)
            def _(g):
                nf16 = plsc.bitcast(
                    normb[pl.ds(ch * _CH + g * 16, 16)], jnp.float32)
                for j in range(16):
                    e = g * 16 + j
                    nb = jnp.full((16,), nf16[j], dtype=jnp.float32)
                    for q in range(dh // 16):
                        rows[e, pl.ds(q * 16, 16)] = (
                            rows[e, pl.ds(q * 16, 16)] * nb)

        def run_pass(table_ref, out_ref, q_off):
            issue_gather(table_ref, 0, 0)
            issue_gather(table_ref, 1, 1)

            @pl.loop(0, (nch + 2) // 3)
            def _(p):
                for lane in range(3):
                    ch = 3 * p + lane

                    @pl.when(ch < nch)
                    def _():
                        wait_gather(table_ref, ch, lane)
                        scale(ch, lane)

                        @pl.when(ch >= 1)
                        def _():
                            wait_scatter(ch - 1, (lane + 2) % 3)

                        @pl.when(ch + 2 < nch)
                        def _():
                            issue_gather(table_ref, ch + 2, (lane + 2) % 3)

                        issue_scatter(ch, lane)

            wait_scatter(nch - 1, (nch - 1) % 3)
            plsc.subcore_barrier()
            pltpu.sync_copy(acc.at[pl.ds(n0, rows_per_tile)],
                            out_ref.at[pl.ds(q_off + n0, rows_per_tile)])
            zero_acc()
            plsc.subcore_barrier()

        run_pass(table0, o1, c * n_pad)
        if npass == 2:
            transform(lambda v: v + 2)
            run_pass(table0, o1, (c + 2) * n_pad)

        sub = 2 * (npass - 1)
        transform(lambda v: ((v - c - sub) >> shift) + c * n_pad)
        for tbl, out in ((o1, o2), (o2, o3)):
            run_pass(tbl, out, c * n_pad)
            if npass == 2:
                transform(lambda v: v + 2 * n_pad)
                run_pass(tbl, out, (c + 2) * n_pad)
                if out is not o3:
                    transform(lambda v: v - 2 * n_pad)

    return props


def _mm_layer(h0, p1, p2, p3, w, b, relu):
    n_pad, d_in = h0.shape
    d_out = w.shape[2]
    a = p1.shape[0]
    bm = 1280
    grid = (n_pad // bm,)

    def body(h0_ref, p1_ref, p2_ref, p3_ref, w_ref, b_ref, o_ref):
        dot = functools.partial(
            jnp.dot, preferred_element_type=jnp.float32,
            precision=jax.lax.Precision.HIGHEST)
        acc = dot(h0_ref[...], w_ref[0])
        for k, pr in ((1, p1_ref), (2, p2_ref), (3, p3_ref)):
            for q in range(a):
                acc += dot(pr[q], w_ref[k, q * 64:(q + 1) * 64, :])
        acc = acc + b_ref[...]
        if relu:
            acc = jnp.maximum(acc, 0.0)
        o_ref[...] = acc

    return pl.pallas_call(
        body,
        grid=grid,
        in_specs=[
            pl.BlockSpec((bm, d_in), lambda i: (i, 0)),
            pl.BlockSpec((a, bm, 64), lambda i: (0, i, 0)),
            pl.BlockSpec((a, bm, 64), lambda i: (0, i, 0)),
            pl.BlockSpec((a, bm, 64), lambda i: (0, i, 0)),
            pl.BlockSpec((4, d_in, d_out), lambda i: (0, 0, 0)),
            pl.BlockSpec((1, d_out), lambda i: (0, 0)),
        ],
        out_specs=pl.BlockSpec((bm, d_out), lambda i: (i, 0)),
        out_shape=jax.ShapeDtypeStruct((n_pad, d_out), jnp.float32),
    )(h0, p1, p2, p3, w, b.reshape(1, d_out))


def kernel(x, edge_index, edge_attr, W1, b1, W2, b2, W3, b3):
    n, d_in = x.shape
    e = edge_index.shape[1]
    n_pad = -(-n // (_NT * _CH)) * (_NT * _CH)

    src = edge_index[0]
    dst = edge_index[1]
    ew = edge_attr.reshape(-1)
    zero_rows = jnp.zeros((_CH, 16), jnp.float32)

    nch = -(-e // (_NT * _CH))
    e_pad = _NT * _CH * nch
    n_chunks = _NT * nch
    srcp = jnp.pad(src, (0, e_pad - e))
    dstp = jnp.pad(dst, (0, e_pad - e))
    pdata = jnp.stack([
        srcp, dstp,
        lax.bitcast_convert_type(jnp.pad(ew, (0, e_pad - e)), jnp.int32),
    ])
    norm_bits = _make_prep(n_pad, e_pad)(pdata, zero_rows)
    edata = jnp.stack([srcp, dstp, norm_bits])

    def layer(h, w, b, relu):
        d = h.shape[1]
        a = d // 64
        zrows = jnp.zeros((_CH, 64), jnp.float32)
        table0 = h.reshape(a * n_pad, 64)
        p1, p2, p3 = _make_layer_props(n_pad, n_chunks, d)(
            table0, edata, zrows)
        return _mm_layer(
            h,
            p1.reshape(a, n_pad, 64),
            p2.reshape(a, n_pad, 64),
            p3.reshape(a, n_pad, 64),
            w, b, relu)

    h = jnp.pad(x, ((0, n_pad - n), (0, 0)))
    h = layer(h, W1, b1, relu=True)
    h = layer(h, W2, b2, relu=True)
    h = layer(h, W3, b3, relu=False)
    return h[:n]
